# initial kernel scaffold (unmeasured)
import jax
import jax.numpy as jnp
from jax import lax
from jax.experimental import pallas as pl
from jax.experimental.pallas import tpu as pltpu


def kernel(x, W, labels):
    T, D = x.shape
    _, V = W.shape
    BV = 2048
    nblk = V // BV

    def body(x_ref, w_ref, l_ref, out_ref, acc_ref, recv_ref, send_sem, recv_sem):
        j = pl.program_id(0)
        my_x = lax.axis_index("x")
        my_y = lax.axis_index("y")
        my_z = lax.axis_index("z")

        logits = jnp.dot(x_ref[:, :], w_ref[:, :],
                         preferred_element_type=jnp.float32)
        bm = jnp.max(logits, axis=1)
        bs = jnp.sum(jnp.exp(logits - bm[:, None]), axis=1)

        offset = my_x * V + j * BV
        loc = l_ref[:] - offset
        cols = lax.broadcasted_iota(jnp.int32, (T, BV), 1)
        lab = jnp.sum(jnp.where(cols == loc[:, None], logits, 0.0), axis=1)

        @pl.when(j == 0)
        def _():
            acc_ref[0, :] = bm
            acc_ref[1, :] = bs
            acc_ref[2, :] = lab

        @pl.when(j > 0)
        def _():
            m_old = acc_ref[0, :]
            s_old = acc_ref[1, :]
            m_new = jnp.maximum(m_old, bm)
            acc_ref[0, :] = m_new
            acc_ref[1, :] = s_old * jnp.exp(m_old - m_new) + bs * jnp.exp(bm - m_new)
            acc_ref[2, :] = acc_ref[2, :] + lab

        @pl.when(j == nblk - 1)
        def _():
            partner = (1 - my_x, my_y, my_z)
            barrier = pltpu.get_barrier_semaphore()
            pl.semaphore_signal(barrier, inc=1, device_id=partner,
                                device_id_type=pl.DeviceIdType.MESH)
            pl.semaphore_wait(barrier, 1)

            rdma = pltpu.make_async_remote_copy(
                src_ref=acc_ref,
                dst_ref=recv_ref,
                send_sem=send_sem,
                recv_sem=recv_sem,
                device_id=partner,
                device_id_type=pl.DeviceIdType.MESH,
            )
            rdma.start()
            rdma.wait()

            m_s, s_s, g_s = acc_ref[0, :], acc_ref[1, :], acc_ref[2, :]
            m_o, s_o, g_o = recv_ref[0, :], recv_ref[1, :], recv_ref[2, :]
            m = jnp.maximum(m_s, m_o)
            s = s_s * jnp.exp(m_s - m) + s_o * jnp.exp(m_o - m)
            out_ref[:] = m + jnp.log(s) - (g_s + g_o)

    return pl.pallas_call(
        body,
        grid=(nblk,),
        out_shape=jax.ShapeDtypeStruct((T,), jnp.float32),
        in_specs=[
            pl.BlockSpec((T, D), lambda j: (0, 0)),
            pl.BlockSpec((D, BV), lambda j: (0, j)),
            pl.BlockSpec((T,), lambda j: (0,)),
        ],
        out_specs=pl.BlockSpec((T,), lambda j: (0,)),
        scratch_shapes=[
            pltpu.VMEM((3, T), jnp.float32),
            pltpu.VMEM((3, T), jnp.float32),
            pltpu.SemaphoreType.DMA,
            pltpu.SemaphoreType.DMA,
        ],
        compiler_params=pltpu.CompilerParams(
            dimension_semantics=("arbitrary",),
            collective_id=0,
        ),
    )(x, W, labels)


# baseline (device time: 109143 ns/iter reference)
import jax
import jax.numpy as jnp
from jax import lax
from jax.experimental import pallas as pl
from jax.experimental.pallas import tpu as pltpu


def kernel(x, W, labels):
    T, D = x.shape
    _, V = W.shape
    BV = 2048
    nblk = V // BV

    def body(x_ref, w_ref, l_ref, out_ref, acc_ref, recv_ref, send_sem, recv_sem):
        j = pl.program_id(0)
        my_x = lax.axis_index("x")
        my_y = lax.axis_index("y")
        my_z = lax.axis_index("z")

        logits = jnp.dot(x_ref[:, :], w_ref[:, :],
                         preferred_element_type=jnp.float32)
        bm = jnp.max(logits, axis=1)
        bs = jnp.sum(jnp.exp(logits - bm[:, None]), axis=1)

        offset = my_x * V + j * BV
        loc = l_ref[:] - offset
        cols = lax.broadcasted_iota(jnp.int32, (T, BV), 1)
        lab = jnp.sum(jnp.where(cols == loc[:, None], logits, 0.0), axis=1)

        @pl.when(j == 0)
        def _():
            acc_ref[0, :] = bm
            acc_ref[1, :] = bs
            acc_ref[2, :] = lab

        @pl.when(j > 0)
        def _():
            m_old = acc_ref[0, :]
            s_old = acc_ref[1, :]
            m_new = jnp.maximum(m_old, bm)
            acc_ref[0, :] = m_new
            acc_ref[1, :] = s_old * jnp.exp(m_old - m_new) + bs * jnp.exp(bm - m_new)
            acc_ref[2, :] = acc_ref[2, :] + lab

        @pl.when(j == nblk - 1)
        def _():
            partner = (1 - my_x, my_y, my_z)
            barrier = pltpu.get_barrier_semaphore()
            pl.semaphore_signal(barrier, inc=1, device_id=partner,
                                device_id_type=pl.DeviceIdType.MESH)
            pl.semaphore_wait(barrier, 1)

            rdma = pltpu.make_async_remote_copy(
                src_ref=acc_ref,
                dst_ref=recv_ref,
                send_sem=send_sem,
                recv_sem=recv_sem,
                device_id=partner,
                device_id_type=pl.DeviceIdType.MESH,
            )
            rdma.start()
            rdma.wait()

            m_s, s_s, g_s = acc_ref[0, :], acc_ref[1, :], acc_ref[2, :]
            m_o, s_o, g_o = recv_ref[0, :], recv_ref[1, :], recv_ref[2, :]
            m = jnp.maximum(m_s, m_o)
            s = s_s * jnp.exp(m_s - m) + s_o * jnp.exp(m_o - m)
            out_ref[:] = m + jnp.log(s) - (g_s + g_o)

    return pl.pallas_call(
        body,
        grid=(nblk,),
        out_shape=jax.ShapeDtypeStruct((T,), jnp.float32),
        in_specs=[
            pl.BlockSpec((T, D), lambda j: (0, 0)),
            pl.BlockSpec((D, BV), lambda j: (0, j)),
            pl.BlockSpec((T,), lambda j: (0,)),
        ],
        out_specs=pl.BlockSpec((T,), lambda j: (0,)),
        scratch_shapes=[
            pltpu.VMEM((3, T), jnp.float32),
            pltpu.VMEM((3, T), jnp.float32),
            pltpu.SemaphoreType.DMA,
            pltpu.SemaphoreType.DMA,
        ],
        compiler_params=pltpu.CompilerParams(
            dimension_semantics=("arbitrary",),
            collective_id=0,
            vmem_limit_bytes=100 * 1024 * 1024,
        ),
    )(x, W, labels)
